# baseline (speedup 1.0000x reference)
import functools

import jax
import jax.numpy as jnp
from jax import lax
from jax.experimental import pallas as pl
from jax.experimental.pallas import tpu as pltpu
from jax.experimental.pallas import tpu_sc as plsc

EPS = 1e-15
N_NODES = 10000
N_EDGES = 320000
FEAT = 128
N_RBF = 20
CUTOFF = 5.0


_BN = 1000


def _phi_body(s_ref, w_ref, b_ref, v16_ref, o_ref):
    phi = (
        jnp.dot(s_ref[...], w_ref[...], preferred_element_type=jnp.float32)
        + b_ref[...]
    )
    lo = jax.lax.bitcast_convert_type(
        phi.astype(jnp.bfloat16), jnp.uint16).astype(jnp.uint32)
    hi = jax.lax.bitcast_convert_type(
        v16_ref[...], jnp.uint16).astype(jnp.uint32)
    o_ref[...] = jax.lax.bitcast_convert_type(lo | (hi << 16), jnp.float32)


def _compute_tab(s_j, W_phi, b_phi, vcat16):
    n = s_j.shape[0]
    grid = n // _BN
    return pl.pallas_call(
        _phi_body,
        grid=(grid,),
        in_specs=[
            pl.BlockSpec((_BN, FEAT), lambda i: (i, 0)),
            pl.BlockSpec((FEAT, 3 * FEAT), lambda i: (0, 0)),
            pl.BlockSpec((1, 3 * FEAT), lambda i: (0, 0)),
            pl.BlockSpec((_BN, 3 * FEAT), lambda i: (i, 0)),
        ],
        out_specs=pl.BlockSpec((_BN, 3 * FEAT), lambda i: (i, 0)),
        out_shape=jax.ShapeDtypeStruct((n, 3 * FEAT), jnp.float32),
    )(s_j, W_phi, b_phi.reshape(1, -1), vcat16)


_BE = 2000
_TABW = 6 * FEAT
_GW = _TABW // 2


def _edge_body(r_ref, rt_ref, tabg_ref, freq_ref, wrbf_ref,
               ds_ref, dvx_ref, dvy_ref, dvz_ref):
    r = r_ref[...]
    d2 = (r * r).sum(axis=1, keepdims=True) + 3.0 * EPS
    dist = jnp.sqrt(d2)
    inv = 1.0 / dist
    rt = rt_ref[...][0]
    d2t = (rt * rt).sum(axis=0, keepdims=True) + 3.0 * EPS
    invt = jax.lax.rsqrt(d2t)
    rbft = jnp.sin(freq_ref[...] * jnp.sqrt(d2t)) * invt
    w_s = jax.lax.dot_general(
        rbft, wrbf_ref[...], (((0,), (0,)), ((), ())),
        preferred_element_type=jnp.float32)
    pw = jax.lax.bitcast_convert_type(tabg_ref[...], jnp.int32)
    phig = jax.lax.bitcast_convert_type(pw << 16, jnp.float32)
    vcat = jax.lax.bitcast_convert_type(
        pw & jnp.int32(-65536), jnp.float32)
    sp0 = phig[:, :FEAT] * w_s[:, :FEAT]
    sp1 = phig[:, FEAT:2 * FEAT] * w_s[:, FEAT:2 * FEAT]
    sp2 = phig[:, 2 * FEAT:] * w_s[:, 2 * FEAT:]
    ds_ref[...] = sp1
    ux = r[:, 0:1] * inv
    uy = r[:, 1:2] * inv
    uz = r[:, 2:3] * inv
    dvx_ref[...] = sp2 * ux + sp0 * vcat[:, :FEAT]
    dvy_ref[...] = sp2 * uy + sp0 * vcat[:, FEAT:2 * FEAT]
    dvz_ref[...] = sp2 * uz + sp0 * vcat[:, 2 * FEAT:]


def _edge_math(r_ij, tabg, W_rbf):
    e = r_ij.shape[0]
    grid = e // _BE
    rt = r_ij.T.reshape(3, grid, _BE).transpose(1, 0, 2)
    freq = (jnp.arange(1, N_RBF + 1, dtype=jnp.float32)
            * (jnp.pi / CUTOFF)).reshape(N_RBF, 1)
    fspec = pl.BlockSpec((_BE, FEAT), lambda i: (i, 0))
    out4 = [jax.ShapeDtypeStruct((e, FEAT), jnp.float32)] * 4
    return pl.pallas_call(
        _edge_body,
        grid=(grid,),
        in_specs=[
            pl.BlockSpec((_BE, 3), lambda i: (i, 0)),
            pl.BlockSpec((1, 3, _BE), lambda i: (i, 0, 0)),
            pl.BlockSpec((_BE, _GW), lambda i: (i, 0)),
            pl.BlockSpec((N_RBF, 1), lambda i: (0, 0)),
            pl.BlockSpec((N_RBF, 3 * FEAT), lambda i: (0, 0)),
        ],
        out_specs=[fspec, fspec, fspec, fspec],
        out_shape=out4,
    )(r_ij, rt, tabg, freq, W_rbf)


_NW = 32
_CH = 80
_GCH = 40
_NSLOT = 5


def _make_gather_body(epw, nch):
    assert nch % _NSLOT == 0 and nch // _NSLOT >= 3

    def body(tab_hbm, dst3_hbm, tabg_hbm, idx_all,
             b0, b1, b2, b3, b4, g0, g1, g2, g3, g4, w0, w1, w2, w3, w4):
        bufs = [b0, b1, b2, b3, b4]
        gsems = [g0, g1, g2, g3, g4]
        wsems = [w0, w1, w2, w3, w4]
        wid = lax.axis_index("s") * 2 + lax.axis_index("c")
        base = wid * epw
        pltpu.sync_copy(dst3_hbm.at[wid], idx_all)

        dummy = tab_hbm.at[pl.ds(0, _GCH)]
        wdummy = tabg_hbm.at[pl.ds(0, _GCH)]

        def start_g(j, s):
            pltpu.async_copy(tab_hbm.at[idx_all.at[j]], bufs[s], gsems[s])

        def step(j, s, jdyn=None):
            jj = j if jdyn is None else jdyn
            pltpu.make_async_copy(dummy, bufs[s], gsems[s]).wait()
            pltpu.async_copy(bufs[s],
                             tabg_hbm.at[pl.ds(base + jj * _GCH, _GCH)],
                             wsems[s])

        start_g(0, 0)
        start_g(1, 1)
        for j in range(_NSLOT):
            step(j, j)
            if j >= 3:
                pltpu.make_async_copy(wdummy, bufs[(j + 2) % _NSLOT],
                                      wsems[(j + 2) % _NSLOT]).wait()
            start_g(j + 2, (j + 2) % _NSLOT)

        def group(q, carry):
            j0 = q * _NSLOT
            for s in range(_NSLOT):
                j = j0 + s
                step(s, s, jdyn=j)
                s2 = (s + 2) % _NSLOT
                pltpu.make_async_copy(wdummy, bufs[s2], wsems[s2]).wait()
                pltpu.async_copy(tab_hbm.at[idx_all.at[j + 2]],
                                 bufs[s2], gsems[s2])
            return carry

        lax.fori_loop(1, nch // _NSLOT - 1, group, 0)

        for j in range(nch - _NSLOT, nch):
            s = j % _NSLOT
            step(j, s)
            pltpu.make_async_copy(wdummy, bufs[(s + 2) % _NSLOT],
                                  wsems[(s + 2) % _NSLOT]).wait()
            if j + 2 < nch:
                start_g(j + 2, (j + 2) % _NSLOT)
        for j in range(nch - 3, nch):
            s = j % _NSLOT
            pltpu.make_async_copy(wdummy, bufs[s], wsems[s]).wait()

    return body


def _sc_gather(tab, dst):
    mesh = plsc.VectorSubcoreMesh(core_axis_name="c", subcore_axis_name="s")
    e = dst.shape[0]
    epw = e // _NW
    nch = epw // _GCH
    dst3 = dst.reshape(_NW, nch, _GCH)
    out_type = jax.ShapeDtypeStruct((e, _GW), jnp.float32)
    f = pl.kernel(
        _make_gather_body(epw, nch),
        out_type=out_type,
        mesh=mesh,
        scratch_types=(
            [pltpu.VMEM((nch, _GCH), jnp.int32)]
            + [pltpu.VMEM((_GCH, _GW), jnp.float32)] * _NSLOT
            + [pltpu.SemaphoreType.DMA] * (2 * _NSLOT)
        ),
    )
    return f(tab, dst3)


_NT = 16
_SCH = 40
_EPT = N_EDGES // _NT
_NCH_S = _EPT // _SCH
_NG = 5
_CPG = _NCH_S // _NG
_SSLOT = 4
_FB = 40
_NFB = N_NODES // _FB


def _scatter_body(ds_hbm, dvx_hbm, dvy_hbm, dvz_hbm, src4_hbm,
                  os_hbm, ovx_hbm, ovy_hbm, ovz_hbm,
                  acc, idx_buf, db0, db1, db2, db3,
                  l0, l1, l2, l3, a0, a1, a2, a3):
    dbufs = [db0, db1, db2, db3]
    lsems = [l0, l1, l2, l3]
    asems = [a0, a1, a2, a3]
    cid = lax.axis_index("c")
    sid = lax.axis_index("s")

    def one_pass(d_hbm, o_hbm):
        def zloop(k, carry):
            db0[k // 8, pl.ds((k % 8) * 16, 16)] = jnp.zeros((16,),
                                                             jnp.float32)
            return carry

        lax.fori_loop(0, _FB * (FEAT // 16), zloop, 0)
        for t in range(-(-_NFB // _NT)):
            b = t * _NT + sid

            @pl.when(b < _NFB)
            def _():
                pltpu.sync_copy(db0, acc.at[pl.ds(b * _FB, _FB)])

        plsc.subcore_barrier()

        dummy = d_hbm.at[pl.ds(0, _SCH)]
        for g in range(_NG):
            pltpu.sync_copy(src4_hbm.at[sid, g], idx_buf)
            gbase = sid * _EPT + g * _CPG * _SCH

            def load(j, s, jdyn=None):
                jj = j if jdyn is None else jdyn
                pltpu.async_copy(
                    d_hbm.at[pl.ds(gbase + jj * _SCH, _SCH)],
                    dbufs[s], lsems[s])

            def add(j, s, jdyn=None):
                jj = j if jdyn is None else jdyn
                pltpu.make_async_copy(dummy, dbufs[s], lsems[s]).wait()
                pltpu.async_copy(dbufs[s], acc.at[idx_buf.at[jj]],
                                 asems[s], add=True)

            load(0, 0)
            load(1, 1)
            for j in range(_SSLOT):
                add(j, j)
                if j >= 2:
                    pltpu.make_async_copy(
                        dummy, dbufs[(j + 2) % _SSLOT],
                        asems[(j + 2) % _SSLOT]).wait()
                load(j + 2, (j + 2) % _SSLOT)

            def body(q, carry):
                j0 = q * _SSLOT
                for s in range(_SSLOT):
                    j = j0 + s
                    add(s, s, jdyn=j)
                    s2 = (s + 2) % _SSLOT
                    pltpu.make_async_copy(dummy, dbufs[s2],
                                          asems[s2]).wait()
                    load(s, s2, jdyn=j + 2)
                return carry

            lax.fori_loop(1, _CPG // _SSLOT - 1, body, 0)

            for j in range(_CPG - _SSLOT, _CPG):
                s = j % _SSLOT
                add(j, s)
                pltpu.make_async_copy(dummy, dbufs[(s + 2) % _SSLOT],
                                      asems[(s + 2) % _SSLOT]).wait()
                if j + 2 < _CPG:
                    load(j + 2, (j + 2) % _SSLOT)
            for j in range(_CPG - 2, _CPG):
                s = j % _SSLOT
                pltpu.make_async_copy(dummy, dbufs[s], asems[s]).wait()

        plsc.subcore_barrier()
        for t in range(-(-_NFB // _NT)):
            b = t * _NT + sid

            @pl.when(b < _NFB)
            def _():
                rows = pl.ds(b * _FB, _FB)
                pltpu.sync_copy(acc.at[rows], o_hbm.at[rows])

        plsc.subcore_barrier()

    @pl.when(cid == 0)
    def _():
        one_pass(ds_hbm, os_hbm)
        one_pass(dvx_hbm, ovx_hbm)

    @pl.when(cid == 1)
    def _():
        one_pass(dvy_hbm, ovy_hbm)
        one_pass(dvz_hbm, ovz_hbm)


def _sc_scatter(ds, dvx, dvy, dvz, src):
    mesh = plsc.VectorSubcoreMesh(core_axis_name="c", subcore_axis_name="s")
    src4 = src.reshape(_NT, _NG, _CPG, _SCH)
    out_type = [jax.ShapeDtypeStruct((N_NODES, FEAT), jnp.float32)] * 4
    f = pl.kernel(
        _scatter_body,
        out_type=out_type,
        mesh=mesh,
        scratch_types=(
            [pltpu.VMEM_SHARED((N_NODES, FEAT), jnp.float32),
             pltpu.VMEM((_CPG, _SCH), jnp.int32)]
            + [pltpu.VMEM((_SCH, FEAT), jnp.float32)] * _SSLOT
            + [pltpu.SemaphoreType.DMA] * (2 * _SSLOT)
        ),
    )
    return f(ds, dvx, dvy, dvz, src4)


def kernel(s_j, v_j, r_ij, nbrs, W_phi, b_phi, W_rbf):
    nbrs = nbrs.astype(jnp.int32)
    src = nbrs[:, 0]
    dst = nbrs[:, 1]
    vt = jnp.transpose(v_j, (2, 0, 1))
    vcat16 = jnp.concatenate([vt[0], vt[1], vt[2]],
                             axis=1).astype(jnp.bfloat16)
    tab32 = _compute_tab(s_j, W_phi, b_phi, vcat16)
    tabg = _sc_gather(tab32, dst)
    ds, dvx, dvy, dvz = _edge_math(r_ij, tabg, W_rbf)
    delta_s, ovx, ovy, ovz = _sc_scatter(ds, dvx, dvy, dvz, src)
    delta_v = jnp.stack([ovx, ovy, ovz], axis=-1)
    return (delta_s, delta_v)


# speedup vs baseline: 1.0653x; 1.0653x over previous
import functools

import jax
import jax.numpy as jnp
from jax import lax
from jax.experimental import pallas as pl
from jax.experimental.pallas import tpu as pltpu
from jax.experimental.pallas import tpu_sc as plsc

EPS = 1e-15
N_NODES = 10000
N_EDGES = 320000
FEAT = 128
N_RBF = 20
CUTOFF = 5.0


_BN = 1000


def _phi_body(s_ref, w_ref, b_ref, v16_ref, o_ref):
    phi = (
        jnp.dot(s_ref[...], w_ref[...], preferred_element_type=jnp.float32)
        + b_ref[...]
    )
    lo = jax.lax.bitcast_convert_type(
        phi.astype(jnp.bfloat16), jnp.uint16).astype(jnp.uint32)
    hi = jax.lax.bitcast_convert_type(
        v16_ref[...], jnp.uint16).astype(jnp.uint32)
    o_ref[...] = jax.lax.bitcast_convert_type(lo | (hi << 16), jnp.float32)


def _compute_tab(s_j, W_phi, b_phi, vcat16):
    n = s_j.shape[0]
    grid = n // _BN
    return pl.pallas_call(
        _phi_body,
        grid=(grid,),
        in_specs=[
            pl.BlockSpec((_BN, FEAT), lambda i: (i, 0)),
            pl.BlockSpec((FEAT, 3 * FEAT), lambda i: (0, 0)),
            pl.BlockSpec((1, 3 * FEAT), lambda i: (0, 0)),
            pl.BlockSpec((_BN, 3 * FEAT), lambda i: (i, 0)),
        ],
        out_specs=pl.BlockSpec((_BN, 3 * FEAT), lambda i: (i, 0)),
        out_shape=jax.ShapeDtypeStruct((n, 3 * FEAT), jnp.float32),
    )(s_j, W_phi, b_phi.reshape(1, -1), vcat16)


_BE = 2000
_TABW = 6 * FEAT
_GW = _TABW // 2


def _edge_body(r_ref, rt_ref, tabg_ref, freq_ref, wrbf_ref,
               ds_ref, dvx_ref, dvy_ref, dvz_ref):
    r = r_ref[...]
    d2 = (r * r).sum(axis=1, keepdims=True) + 3.0 * EPS
    dist = jnp.sqrt(d2)
    inv = 1.0 / dist
    rt = rt_ref[...][0]
    d2t = (rt * rt).sum(axis=0, keepdims=True) + 3.0 * EPS
    invt = jax.lax.rsqrt(d2t)
    rbft = jnp.sin(freq_ref[...] * jnp.sqrt(d2t)) * invt
    w_s = jax.lax.dot_general(
        rbft, wrbf_ref[...], (((0,), (0,)), ((), ())),
        preferred_element_type=jnp.float32)
    pw = jax.lax.bitcast_convert_type(tabg_ref[...], jnp.int32)
    phig = jax.lax.bitcast_convert_type(pw << 16, jnp.float32)
    vcat = jax.lax.bitcast_convert_type(
        pw & jnp.int32(-65536), jnp.float32)
    sp0 = phig[:, :FEAT] * w_s[:, :FEAT]
    sp1 = phig[:, FEAT:2 * FEAT] * w_s[:, FEAT:2 * FEAT]
    sp2 = phig[:, 2 * FEAT:] * w_s[:, 2 * FEAT:]
    ds_ref[...] = sp1
    ux = r[:, 0:1] * inv
    uy = r[:, 1:2] * inv
    uz = r[:, 2:3] * inv
    dvx_ref[...] = sp2 * ux + sp0 * vcat[:, :FEAT]
    dvy_ref[...] = sp2 * uy + sp0 * vcat[:, FEAT:2 * FEAT]
    dvz_ref[...] = sp2 * uz + sp0 * vcat[:, 2 * FEAT:]


def _edge_math(r_ij, tabg, W_rbf):
    e = r_ij.shape[0]
    grid = e // _BE
    rt = r_ij.T.reshape(3, grid, _BE).transpose(1, 0, 2)
    freq = (jnp.arange(1, N_RBF + 1, dtype=jnp.float32)
            * (jnp.pi / CUTOFF)).reshape(N_RBF, 1)
    fspec = pl.BlockSpec((_BE, FEAT), lambda i: (i, 0))
    out4 = [jax.ShapeDtypeStruct((e, FEAT), jnp.float32)] * 4
    return pl.pallas_call(
        _edge_body,
        grid=(grid,),
        in_specs=[
            pl.BlockSpec((_BE, 3), lambda i: (i, 0)),
            pl.BlockSpec((1, 3, _BE), lambda i: (i, 0, 0)),
            pl.BlockSpec((_BE, _GW), lambda i: (i, 0)),
            pl.BlockSpec((N_RBF, 1), lambda i: (0, 0)),
            pl.BlockSpec((N_RBF, 3 * FEAT), lambda i: (0, 0)),
        ],
        out_specs=[fspec, fspec, fspec, fspec],
        out_shape=out4,
    )(r_ij, rt, tabg, freq, W_rbf)


_NW = 32
_CH = 80
_GCH = 40
_NSLOT = 5


def _make_gather_body(epw, nch):
    assert nch % _NSLOT == 0

    def body(tab_hbm, dst3_hbm, tabg_hbm, idx_all,
             b0, b1, b2, b3, b4, s0, s1, s2, s3, s4):
        bufs = [b0, b1, b2, b3, b4]
        sems = [s0, s1, s2, s3, s4]
        wid = lax.axis_index("s") * 2 + lax.axis_index("c")
        base = wid * epw
        pltpu.sync_copy(dst3_hbm.at[wid], idx_all)

        dummy = tab_hbm.at[pl.ds(0, _GCH)]
        for s in range(_NSLOT):
            pltpu.async_copy(tab_hbm.at[idx_all.at[s]], bufs[s], sems[s])

        def group(q, carry):
            j0 = q * _NSLOT
            for s in range(_NSLOT):
                j = j0 + s
                pltpu.make_async_copy(dummy, bufs[s], sems[s]).wait()
                pltpu.sync_copy(bufs[s],
                                tabg_hbm.at[pl.ds(base + j * _GCH, _GCH)])

                @pl.when(j + _NSLOT < nch)
                def _(s=s, j=j):
                    pltpu.async_copy(tab_hbm.at[idx_all.at[j + _NSLOT]],
                                     bufs[s], sems[s])

            return carry

        lax.fori_loop(0, nch // _NSLOT, group, 0)

    return body


def _sc_gather(tab, dst):
    mesh = plsc.VectorSubcoreMesh(core_axis_name="c", subcore_axis_name="s")
    e = dst.shape[0]
    epw = e // _NW
    nch = epw // _GCH
    dst3 = dst.reshape(_NW, nch, _GCH)
    out_type = jax.ShapeDtypeStruct((e, _GW), jnp.float32)
    f = pl.kernel(
        _make_gather_body(epw, nch),
        out_type=out_type,
        mesh=mesh,
        scratch_types=(
            [pltpu.VMEM((nch, _GCH), jnp.int32)]
            + [pltpu.VMEM((_GCH, _GW), jnp.float32)] * _NSLOT
            + [pltpu.SemaphoreType.DMA] * _NSLOT
        ),
    )
    return f(tab, dst3)


_NT = 16
_EPT = N_EDGES // _NT
_NCH_S = _EPT // _CH
_NG = 5
_CPG = _NCH_S // _NG
_FB = 80
_NFB = N_NODES // _FB


def _scatter_body(ds_hbm, dvx_hbm, dvy_hbm, dvz_hbm, src4_hbm,
                  os_hbm, ovx_hbm, ovy_hbm, ovz_hbm,
                  acc, idx_buf, dbuf0, dbuf1, sem0, sem1):
    cid = lax.axis_index("c")
    sid = lax.axis_index("s")

    def one_pass(d_hbm, o_hbm):
        def zloop(k, carry):
            dbuf0[k // 8, pl.ds((k % 8) * 16, 16)] = jnp.zeros((16,),
                                                               jnp.float32)
            return carry

        lax.fori_loop(0, _FB * (FEAT // 16), zloop, 0)
        for t in range(-(-_NFB // _NT)):
            b = t * _NT + sid

            @pl.when(b < _NFB)
            def _():
                pltpu.sync_copy(dbuf0, acc.at[pl.ds(b * _FB, _FB)])

        plsc.subcore_barrier()

        dummy = d_hbm.at[pl.ds(0, _CH)]
        for g in range(_NG):
            pltpu.sync_copy(src4_hbm.at[sid, g], idx_buf)
            gbase = sid * _EPT + g * _CPG * _CH
            pltpu.async_copy(d_hbm.at[pl.ds(gbase, _CH)], dbuf0, sem0)

            def pair(p, carry, gbase=gbase):
                j0 = 2 * p
                j1 = j0 + 1
                pltpu.async_copy(d_hbm.at[pl.ds(gbase + j1 * _CH, _CH)],
                                 dbuf1, sem1)
                pltpu.make_async_copy(dummy, dbuf0, sem0).wait()
                pltpu.sync_copy(dbuf0, acc.at[idx_buf.at[j0]], add=True)

                @pl.when(j1 + 1 < _CPG)
                def _():
                    pltpu.async_copy(
                        d_hbm.at[pl.ds(gbase + (j1 + 1) * _CH, _CH)],
                        dbuf0, sem0)

                pltpu.make_async_copy(dummy, dbuf1, sem1).wait()
                pltpu.sync_copy(dbuf1, acc.at[idx_buf.at[j1]], add=True)
                return carry

            lax.fori_loop(0, _CPG // 2, pair, 0)
        plsc.subcore_barrier()
        for t in range(-(-_NFB // _NT)):
            b = t * _NT + sid

            @pl.when(b < _NFB)
            def _():
                rows = pl.ds(b * _FB, _FB)
                pltpu.sync_copy(acc.at[rows], o_hbm.at[rows])

        plsc.subcore_barrier()

    @pl.when(cid == 0)
    def _():
        one_pass(ds_hbm, os_hbm)
        one_pass(dvx_hbm, ovx_hbm)

    @pl.when(cid == 1)
    def _():
        one_pass(dvy_hbm, ovy_hbm)
        one_pass(dvz_hbm, ovz_hbm)


def _sc_scatter(ds, dvx, dvy, dvz, src):
    mesh = plsc.VectorSubcoreMesh(core_axis_name="c", subcore_axis_name="s")
    src4 = src.reshape(_NT, _NG, _CPG, _CH)
    out_type = [jax.ShapeDtypeStruct((N_NODES, FEAT), jnp.float32)] * 4
    f = pl.kernel(
        _scatter_body,
        out_type=out_type,
        mesh=mesh,
        scratch_types=[
            pltpu.VMEM_SHARED((N_NODES, FEAT), jnp.float32),
            pltpu.VMEM((_CPG, _CH), jnp.int32),
            pltpu.VMEM((_CH, FEAT), jnp.float32),
            pltpu.VMEM((_CH, FEAT), jnp.float32),
            pltpu.SemaphoreType.DMA,
            pltpu.SemaphoreType.DMA,
        ],
    )
    return f(ds, dvx, dvy, dvz, src4)


def kernel(s_j, v_j, r_ij, nbrs, W_phi, b_phi, W_rbf):
    nbrs = nbrs.astype(jnp.int32)
    src = nbrs[:, 0]
    dst = nbrs[:, 1]
    vt = jnp.transpose(v_j, (2, 0, 1))
    vcat16 = jnp.concatenate([vt[0], vt[1], vt[2]],
                             axis=1).astype(jnp.bfloat16)
    tab32 = _compute_tab(s_j, W_phi, b_phi, vcat16)
    tabg = _sc_gather(tab32, dst)
    ds, dvx, dvy, dvz = _edge_math(r_ij, tabg, W_rbf)
    delta_s, ovx, ovy, ovz = _sc_scatter(ds, dvx, dvy, dvz, src)
    delta_v = jnp.stack([ovx, ovy, ovz], axis=-1)
    return (delta_s, delta_v)


# speedup vs baseline: 1.0703x; 1.0047x over previous
import functools

import jax
import jax.numpy as jnp
from jax import lax
from jax.experimental import pallas as pl
from jax.experimental.pallas import tpu as pltpu
from jax.experimental.pallas import tpu_sc as plsc

EPS = 1e-15
N_NODES = 10000
N_EDGES = 320000
FEAT = 128
N_RBF = 20
CUTOFF = 5.0


_BN = 1000


def _phi_body(s_ref, w_ref, b_ref, v16_ref, o_ref):
    phi = (
        jnp.dot(s_ref[...], w_ref[...], preferred_element_type=jnp.float32)
        + b_ref[...]
    )
    lo = jax.lax.bitcast_convert_type(
        phi.astype(jnp.bfloat16), jnp.uint16).astype(jnp.uint32)
    hi = jax.lax.bitcast_convert_type(
        v16_ref[...], jnp.uint16).astype(jnp.uint32)
    o_ref[...] = jax.lax.bitcast_convert_type(lo | (hi << 16), jnp.float32)


def _compute_tab(s_j, W_phi, b_phi, vcat16):
    n = s_j.shape[0]
    grid = n // _BN
    return pl.pallas_call(
        _phi_body,
        grid=(grid,),
        in_specs=[
            pl.BlockSpec((_BN, FEAT), lambda i: (i, 0)),
            pl.BlockSpec((FEAT, 3 * FEAT), lambda i: (0, 0)),
            pl.BlockSpec((1, 3 * FEAT), lambda i: (0, 0)),
            pl.BlockSpec((_BN, 3 * FEAT), lambda i: (i, 0)),
        ],
        out_specs=pl.BlockSpec((_BN, 3 * FEAT), lambda i: (i, 0)),
        out_shape=jax.ShapeDtypeStruct((n, 3 * FEAT), jnp.float32),
    )(s_j, W_phi, b_phi.reshape(1, -1), vcat16)


_BE = 2000
_TABW = 6 * FEAT
_GW = _TABW // 2


def _edge_body(r_ref, rt_ref, tabg_ref, freq_ref, wrbf_ref,
               ds_ref, dvx_ref, dvy_ref, dvz_ref):
    r = r_ref[...]
    d2 = (r * r).sum(axis=1, keepdims=True) + 3.0 * EPS
    dist = jnp.sqrt(d2)
    inv = 1.0 / dist
    rt = rt_ref[...][0]
    d2t = (rt * rt).sum(axis=0, keepdims=True) + 3.0 * EPS
    invt = jax.lax.rsqrt(d2t)
    rbft = jnp.sin(freq_ref[...] * jnp.sqrt(d2t)) * invt
    w_s = jax.lax.dot_general(
        rbft, wrbf_ref[...], (((0,), (0,)), ((), ())),
        preferred_element_type=jnp.float32)
    pw = jax.lax.bitcast_convert_type(tabg_ref[...], jnp.int32)
    phig = jax.lax.bitcast_convert_type(pw << 16, jnp.float32)
    vcat = jax.lax.bitcast_convert_type(
        pw & jnp.int32(-65536), jnp.float32)
    sp0 = phig[:, :FEAT] * w_s[:, :FEAT]
    sp1 = phig[:, FEAT:2 * FEAT] * w_s[:, FEAT:2 * FEAT]
    sp2 = phig[:, 2 * FEAT:] * w_s[:, 2 * FEAT:]
    ds_ref[...] = sp1
    ux = r[:, 0:1] * inv
    uy = r[:, 1:2] * inv
    uz = r[:, 2:3] * inv
    dvx_ref[...] = sp2 * ux + sp0 * vcat[:, :FEAT]
    dvy_ref[...] = sp2 * uy + sp0 * vcat[:, FEAT:2 * FEAT]
    dvz_ref[...] = sp2 * uz + sp0 * vcat[:, 2 * FEAT:]


def _edge_math(r_ij, tabg, W_rbf):
    e = r_ij.shape[0]
    grid = e // _BE
    rt = r_ij.T.reshape(3, grid, _BE).transpose(1, 0, 2)
    freq = (jnp.arange(1, N_RBF + 1, dtype=jnp.float32)
            * (jnp.pi / CUTOFF)).reshape(N_RBF, 1)
    fspec = pl.BlockSpec((_BE, FEAT), lambda i: (i, 0))
    out4 = [jax.ShapeDtypeStruct((e, FEAT), jnp.float32)] * 4
    return pl.pallas_call(
        _edge_body,
        grid=(grid,),
        in_specs=[
            pl.BlockSpec((_BE, 3), lambda i: (i, 0)),
            pl.BlockSpec((1, 3, _BE), lambda i: (i, 0, 0)),
            pl.BlockSpec((_BE, _GW), lambda i: (i, 0)),
            pl.BlockSpec((N_RBF, 1), lambda i: (0, 0)),
            pl.BlockSpec((N_RBF, 3 * FEAT), lambda i: (0, 0)),
        ],
        out_specs=[fspec, fspec, fspec, fspec],
        out_shape=out4,
    )(r_ij, rt, tabg, freq, W_rbf)


_NW = 32
_CH = 80
_GCH = 40
_NSLOT = 5


def _make_gather_body(epw, nch):
    assert nch % _NSLOT == 0

    def body(tab_hbm, dst3_hbm, tabg_hbm, idx_all,
             b0, b1, b2, b3, b4, s0, s1, s2, s3, s4):
        bufs = [b0, b1, b2, b3, b4]
        sems = [s0, s1, s2, s3, s4]
        wid = lax.axis_index("s") * 2 + lax.axis_index("c")
        base = wid * epw
        pltpu.sync_copy(dst3_hbm.at[wid], idx_all)

        dummy = tab_hbm.at[pl.ds(0, _GCH)]
        for s in range(_NSLOT):
            pltpu.async_copy(tab_hbm.at[idx_all.at[s]], bufs[s], sems[s])

        def group(q, carry):
            j0 = q * _NSLOT
            for s in range(_NSLOT):
                j = j0 + s
                pltpu.make_async_copy(dummy, bufs[s], sems[s]).wait()
                pltpu.sync_copy(bufs[s],
                                tabg_hbm.at[pl.ds(base + j * _GCH, _GCH)])

                @pl.when(j + _NSLOT < nch)
                def _(s=s, j=j):
                    pltpu.async_copy(tab_hbm.at[idx_all.at[j + _NSLOT]],
                                     bufs[s], sems[s])

            return carry

        lax.fori_loop(0, nch // _NSLOT, group, 0)

    return body


def _sc_gather(tab, dst):
    mesh = plsc.VectorSubcoreMesh(core_axis_name="c", subcore_axis_name="s")
    e = dst.shape[0]
    epw = e // _NW
    nch = epw // _GCH
    dst3 = dst.reshape(_NW, nch, _GCH)
    out_type = jax.ShapeDtypeStruct((e, _GW), jnp.float32)
    f = pl.kernel(
        _make_gather_body(epw, nch),
        out_type=out_type,
        mesh=mesh,
        scratch_types=(
            [pltpu.VMEM((nch, _GCH), jnp.int32)]
            + [pltpu.VMEM((_GCH, _GW), jnp.float32)] * _NSLOT
            + [pltpu.SemaphoreType.DMA] * _NSLOT
        ),
    )
    return f(tab, dst3)


_NT = 16
_EPT = N_EDGES // _NT
_NCH_S = _EPT // _CH
_NG = 5
_CPG = _NCH_S // _NG
_FB = 80
_NFB = N_NODES // _FB


_SSLOT = 4


def _scatter_body(ds_hbm, dvx_hbm, dvy_hbm, dvz_hbm, src4_hbm,
                  os_hbm, ovx_hbm, ovy_hbm, ovz_hbm,
                  acc, idx_buf, db0, db1, db2, db3,
                  l0, l1, l2, l3, a0, a1, a2, a3):
    dbufs = [db0, db1, db2, db3]
    lsems = [l0, l1, l2, l3]
    asems = [a0, a1, a2, a3]
    cid = lax.axis_index("c")
    sid = lax.axis_index("s")

    def one_pass(d_hbm, o_hbm):
        def zloop(k, carry):
            db0[k // 8, pl.ds((k % 8) * 16, 16)] = jnp.zeros((16,),
                                                             jnp.float32)
            return carry

        lax.fori_loop(0, _FB * (FEAT // 16), zloop, 0)
        for t in range(-(-_NFB // _NT)):
            b = t * _NT + sid

            @pl.when(b < _NFB)
            def _():
                pltpu.sync_copy(db0, acc.at[pl.ds(b * _FB, _FB)])

        plsc.subcore_barrier()

        dummy = d_hbm.at[pl.ds(0, _CH)]
        nmain = _CPG - (_CPG % _SSLOT)
        for g in range(_NG):
            pltpu.sync_copy(src4_hbm.at[sid, g], idx_buf)
            gbase = sid * _EPT + g * _CPG * _CH

            def load(s, jj):
                pltpu.async_copy(
                    d_hbm.at[pl.ds(gbase + jj * _CH, _CH)],
                    dbufs[s], lsems[s])

            def add(s, jj):
                pltpu.make_async_copy(dummy, dbufs[s], lsems[s]).wait()
                pltpu.async_copy(dbufs[s], acc.at[idx_buf.at[jj]],
                                 asems[s], add=True)

            def wait_add(s):
                pltpu.make_async_copy(dummy, dbufs[s], asems[s]).wait()

            load(0, 0)
            load(1, 1)
            for j in range(_SSLOT):
                add(j, j)
                if j >= 2:
                    wait_add((j + 2) % _SSLOT)
                load((j + 2) % _SSLOT, j + 2)

            def body(q, carry):
                j0 = q * _SSLOT
                for s in range(_SSLOT):
                    j = j0 + s
                    add(s, j)
                    s2 = (s + 2) % _SSLOT
                    wait_add(s2)
                    load(s2, j + 2)
                return carry

            lax.fori_loop(1, nmain // _SSLOT - 1, body, 0)

            for j in range(nmain - _SSLOT, _CPG):
                s = j % _SSLOT
                add(s, j)
                wait_add((s + 2) % _SSLOT)
                if j + 2 < _CPG:
                    load((j + 2) % _SSLOT, j + 2)
            for j in range(_CPG - 2, _CPG):
                wait_add(j % _SSLOT)

        plsc.subcore_barrier()
        for t in range(-(-_NFB // _NT)):
            b = t * _NT + sid

            @pl.when(b < _NFB)
            def _():
                rows = pl.ds(b * _FB, _FB)
                pltpu.sync_copy(acc.at[rows], o_hbm.at[rows])

        plsc.subcore_barrier()

    @pl.when(cid == 0)
    def _():
        one_pass(ds_hbm, os_hbm)
        one_pass(dvx_hbm, ovx_hbm)

    @pl.when(cid == 1)
    def _():
        one_pass(dvy_hbm, ovy_hbm)
        one_pass(dvz_hbm, ovz_hbm)


def _sc_scatter(ds, dvx, dvy, dvz, src):
    mesh = plsc.VectorSubcoreMesh(core_axis_name="c", subcore_axis_name="s")
    src4 = src.reshape(_NT, _NG, _CPG, _CH)
    out_type = [jax.ShapeDtypeStruct((N_NODES, FEAT), jnp.float32)] * 4
    f = pl.kernel(
        _scatter_body,
        out_type=out_type,
        mesh=mesh,
        scratch_types=(
            [pltpu.VMEM_SHARED((N_NODES, FEAT), jnp.float32),
             pltpu.VMEM((_CPG, _CH), jnp.int32)]
            + [pltpu.VMEM((_CH, FEAT), jnp.float32)] * _SSLOT
            + [pltpu.SemaphoreType.DMA] * (2 * _SSLOT)
        ),
    )
    return f(ds, dvx, dvy, dvz, src4)


def kernel(s_j, v_j, r_ij, nbrs, W_phi, b_phi, W_rbf):
    nbrs = nbrs.astype(jnp.int32)
    src = nbrs[:, 0]
    dst = nbrs[:, 1]
    vt = jnp.transpose(v_j, (2, 0, 1))
    vcat16 = jnp.concatenate([vt[0], vt[1], vt[2]],
                             axis=1).astype(jnp.bfloat16)
    tab32 = _compute_tab(s_j, W_phi, b_phi, vcat16)
    tabg = _sc_gather(tab32, dst)
    ds, dvx, dvy, dvz = _edge_math(r_ij, tabg, W_rbf)
    delta_s, ovx, ovy, ovz = _sc_scatter(ds, dvx, dvy, dvz, src)
    delta_v = jnp.stack([ovx, ovy, ovz], axis=-1)
    return (delta_s, delta_v)


# speedup vs baseline: 1.0704x; 1.0001x over previous
import jax
import jax.numpy as jnp
from jax import lax
from jax.experimental import pallas as pl
from jax.experimental.pallas import tpu as pltpu
from jax.experimental.pallas import tpu_sc as plsc

EPS = 1e-15
N_NODES = 10000
N_EDGES = 320000
FEAT = 128
N_RBF = 20
CUTOFF = 5.0


_BN = 1000


def _phi_body(s_ref, w_ref, b_ref, v16_ref, o_ref):
    phi = (
        jnp.dot(s_ref[...], w_ref[...], preferred_element_type=jnp.float32)
        + b_ref[...]
    )
    lo = jax.lax.bitcast_convert_type(
        phi.astype(jnp.bfloat16), jnp.uint16).astype(jnp.uint32)
    hi = jax.lax.bitcast_convert_type(
        v16_ref[...], jnp.uint16).astype(jnp.uint32)
    o_ref[...] = jax.lax.bitcast_convert_type(lo | (hi << 16), jnp.float32)


def _compute_tab(s_j, W_phi, b_phi, vcat16):
    n = s_j.shape[0]
    grid = n // _BN
    return pl.pallas_call(
        _phi_body,
        grid=(grid,),
        in_specs=[
            pl.BlockSpec((_BN, FEAT), lambda i: (i, 0)),
            pl.BlockSpec((FEAT, 3 * FEAT), lambda i: (0, 0)),
            pl.BlockSpec((1, 3 * FEAT), lambda i: (0, 0)),
            pl.BlockSpec((_BN, 3 * FEAT), lambda i: (i, 0)),
        ],
        out_specs=pl.BlockSpec((_BN, 3 * FEAT), lambda i: (i, 0)),
        out_shape=jax.ShapeDtypeStruct((n, 3 * FEAT), jnp.float32),
    )(s_j, W_phi, b_phi.reshape(1, -1), vcat16)


_BE = 2000
_TABW = 6 * FEAT
_GW = _TABW // 2


def _edge_body(r_ref, rt_ref, tabg_ref, freq_ref, wrbf_ref,
               ds_ref, dvx_ref, dvy_ref, dvz_ref):
    r = r_ref[...]
    d2 = (r * r).sum(axis=1, keepdims=True) + 3.0 * EPS
    dist = jnp.sqrt(d2)
    inv = 1.0 / dist
    rt = rt_ref[...][0]
    d2t = (rt * rt).sum(axis=0, keepdims=True) + 3.0 * EPS
    invt = jax.lax.rsqrt(d2t)
    rbft = jnp.sin(freq_ref[...] * jnp.sqrt(d2t)) * invt
    w_s = jax.lax.dot_general(
        rbft, wrbf_ref[...], (((0,), (0,)), ((), ())),
        preferred_element_type=jnp.float32)
    pw = jax.lax.bitcast_convert_type(tabg_ref[...], jnp.int32)
    phig = jax.lax.bitcast_convert_type(pw << 16, jnp.float32)
    vcat = jax.lax.bitcast_convert_type(
        pw & jnp.int32(-65536), jnp.float32)
    sp0 = phig[:, :FEAT] * w_s[:, :FEAT]
    sp1 = phig[:, FEAT:2 * FEAT] * w_s[:, FEAT:2 * FEAT]
    sp2 = phig[:, 2 * FEAT:] * w_s[:, 2 * FEAT:]
    ds_ref[...] = sp1
    ux = r[:, 0:1] * inv
    uy = r[:, 1:2] * inv
    uz = r[:, 2:3] * inv
    dvx_ref[...] = sp2 * ux + sp0 * vcat[:, :FEAT]
    dvy_ref[...] = sp2 * uy + sp0 * vcat[:, FEAT:2 * FEAT]
    dvz_ref[...] = sp2 * uz + sp0 * vcat[:, 2 * FEAT:]


def _edge_math(r_ij, tabg, W_rbf):
    e = r_ij.shape[0]
    grid = e // _BE
    rt = r_ij.T.reshape(3, grid, _BE).transpose(1, 0, 2)
    freq = (jnp.arange(1, N_RBF + 1, dtype=jnp.float32)
            * (jnp.pi / CUTOFF)).reshape(N_RBF, 1)
    fspec = pl.BlockSpec((_BE, FEAT), lambda i: (i, 0))
    out4 = [jax.ShapeDtypeStruct((e, FEAT), jnp.float32)] * 4
    return pl.pallas_call(
        _edge_body,
        grid=(grid,),
        in_specs=[
            pl.BlockSpec((_BE, 3), lambda i: (i, 0)),
            pl.BlockSpec((1, 3, _BE), lambda i: (i, 0, 0)),
            pl.BlockSpec((_BE, _GW), lambda i: (i, 0)),
            pl.BlockSpec((N_RBF, 1), lambda i: (0, 0)),
            pl.BlockSpec((N_RBF, 3 * FEAT), lambda i: (0, 0)),
        ],
        out_specs=[fspec, fspec, fspec, fspec],
        out_shape=out4,
    )(r_ij, rt, tabg, freq, W_rbf)


_NW = 32
_CH = 80
_GCH = 40
_NSLOT = 5


def _make_gather_body(epw, nch):
    assert nch % _NSLOT == 0

    def body(tab_hbm, dst3_hbm, tabg_hbm, idx_all,
             b0, b1, b2, b3, b4, s0, s1, s2, s3, s4):
        bufs = [b0, b1, b2, b3, b4]
        sems = [s0, s1, s2, s3, s4]
        wid = lax.axis_index("s") * 2 + lax.axis_index("c")
        base = wid * epw
        pltpu.sync_copy(dst3_hbm.at[wid], idx_all)

        dummy = tab_hbm.at[pl.ds(0, _GCH)]
        for s in range(_NSLOT):
            pltpu.async_copy(tab_hbm.at[idx_all.at[s]], bufs[s], sems[s])

        def group(q, carry):
            j0 = q * _NSLOT
            for s in range(_NSLOT):
                j = j0 + s
                pltpu.make_async_copy(dummy, bufs[s], sems[s]).wait()
                pltpu.sync_copy(bufs[s],
                                tabg_hbm.at[pl.ds(base + j * _GCH, _GCH)])

                @pl.when(j + _NSLOT < nch)
                def _(s=s, j=j):
                    pltpu.async_copy(tab_hbm.at[idx_all.at[j + _NSLOT]],
                                     bufs[s], sems[s])

            return carry

        lax.fori_loop(0, nch // _NSLOT, group, 0)

    return body


def _sc_gather(tab, dst):
    mesh = plsc.VectorSubcoreMesh(core_axis_name="c", subcore_axis_name="s")
    e = dst.shape[0]
    epw = e // _NW
    nch = epw // _GCH
    dst3 = dst.reshape(_NW, nch, _GCH)
    out_type = jax.ShapeDtypeStruct((e, _GW), jnp.float32)
    f = pl.kernel(
        _make_gather_body(epw, nch),
        out_type=out_type,
        mesh=mesh,
        scratch_types=(
            [pltpu.VMEM((nch, _GCH), jnp.int32)]
            + [pltpu.VMEM((_GCH, _GW), jnp.float32)] * _NSLOT
            + [pltpu.SemaphoreType.DMA] * _NSLOT
        ),
    )
    return f(tab, dst3)


_NT = 16
_EPT = N_EDGES // _NT
_NCH_S = _EPT // _CH
_NG = 5
_CPG = _NCH_S // _NG
_FB = 80
_NFB = N_NODES // _FB


_SSLOT = 4


def _scatter_body(ds_hbm, dvx_hbm, dvy_hbm, dvz_hbm, src4_hbm,
                  os_hbm, ovx_hbm, ovy_hbm, ovz_hbm,
                  acc, idx_buf, db0, db1, db2, db3,
                  l0, l1, l2, l3, a0, a1, a2, a3):
    dbufs = [db0, db1, db2, db3]
    lsems = [l0, l1, l2, l3]
    asems = [a0, a1, a2, a3]
    cid = lax.axis_index("c")
    sid = lax.axis_index("s")

    def one_pass(d_hbm, o_hbm):
        def zloop(k, carry):
            db0[k // 8, pl.ds((k % 8) * 16, 16)] = jnp.zeros((16,),
                                                             jnp.float32)
            return carry

        lax.fori_loop(0, _FB * (FEAT // 16), zloop, 0)
        for t in range(-(-_NFB // _NT)):
            b = t * _NT + sid

            @pl.when(b < _NFB)
            def _():
                pltpu.sync_copy(db0, acc.at[pl.ds(b * _FB, _FB)])

        plsc.subcore_barrier()

        dummy = d_hbm.at[pl.ds(0, _CH)]
        nmain = _CPG - (_CPG % _SSLOT)
        for g in range(_NG):
            pltpu.sync_copy(src4_hbm.at[sid, g], idx_buf)
            gbase = sid * _EPT + g * _CPG * _CH

            def load(s, jj):
                pltpu.async_copy(
                    d_hbm.at[pl.ds(gbase + jj * _CH, _CH)],
                    dbufs[s], lsems[s])

            def add(s, jj):
                pltpu.make_async_copy(dummy, dbufs[s], lsems[s]).wait()
                pltpu.async_copy(dbufs[s], acc.at[idx_buf.at[jj]],
                                 asems[s], add=True)

            def wait_add(s):
                pltpu.make_async_copy(dummy, dbufs[s], asems[s]).wait()

            load(0, 0)
            load(1, 1)
            for j in range(_SSLOT):
                add(j, j)
                if j >= 2:
                    wait_add((j + 2) % _SSLOT)
                load((j + 2) % _SSLOT, j + 2)

            def body(q, carry):
                j0 = q * _SSLOT
                for s in range(_SSLOT):
                    j = j0 + s
                    add(s, j)
                    s2 = (s + 2) % _SSLOT
                    wait_add(s2)
                    load(s2, j + 2)
                return carry

            lax.fori_loop(1, nmain // _SSLOT - 1, body, 0)

            for j in range(nmain - _SSLOT, _CPG):
                s = j % _SSLOT
                add(s, j)
                wait_add((s + 2) % _SSLOT)
                if j + 2 < _CPG:
                    load((j + 2) % _SSLOT, j + 2)
            for j in range(_CPG - 2, _CPG):
                wait_add(j % _SSLOT)

        plsc.subcore_barrier()
        for t in range(-(-_NFB // _NT)):
            b = t * _NT + sid

            @pl.when(b < _NFB)
            def _():
                rows = pl.ds(b * _FB, _FB)
                pltpu.sync_copy(acc.at[rows], o_hbm.at[rows])

        plsc.subcore_barrier()

    @pl.when(cid == 0)
    def _():
        one_pass(ds_hbm, os_hbm)
        one_pass(dvx_hbm, ovx_hbm)

    @pl.when(cid == 1)
    def _():
        one_pass(dvy_hbm, ovy_hbm)
        one_pass(dvz_hbm, ovz_hbm)


def _sc_scatter(ds, dvx, dvy, dvz, src):
    mesh = plsc.VectorSubcoreMesh(core_axis_name="c", subcore_axis_name="s")
    src4 = src.reshape(_NT, _NG, _CPG, _CH)
    out_type = [jax.ShapeDtypeStruct((N_NODES, FEAT), jnp.float32)] * 4
    f = pl.kernel(
        _scatter_body,
        out_type=out_type,
        mesh=mesh,
        scratch_types=(
            [pltpu.VMEM_SHARED((N_NODES, FEAT), jnp.float32),
             pltpu.VMEM((_CPG, _CH), jnp.int32)]
            + [pltpu.VMEM((_CH, FEAT), jnp.float32)] * _SSLOT
            + [pltpu.SemaphoreType.DMA] * (2 * _SSLOT)
        ),
    )
    return f(ds, dvx, dvy, dvz, src4)


def kernel(s_j, v_j, r_ij, nbrs, W_phi, b_phi, W_rbf):
    nbrs = nbrs.astype(jnp.int32)
    src = nbrs[:, 0]
    dst = nbrs[:, 1]
    vt = jnp.transpose(v_j, (2, 0, 1))
    vcat16 = jnp.concatenate([vt[0], vt[1], vt[2]],
                             axis=1).astype(jnp.bfloat16)
    tab32 = _compute_tab(s_j, W_phi, b_phi, vcat16)
    tabg = _sc_gather(tab32, dst)
    ds, dvx, dvy, dvz = _edge_math(r_ij, tabg, W_rbf)
    delta_s, ovx, ovy, ovz = _sc_scatter(ds, dvx, dvy, dvz, src)
    delta_v = jnp.stack([ovx, ovy, ovz], axis=-1)
    return (delta_s, delta_v)
